# baseline (speedup 1.0000x reference)
import jax
import jax.numpy as jnp
from jax.experimental import pallas as pl
from jax.experimental.pallas import tpu as pltpu

_SOLVE_ITERS = 8
_ROW_BLOCK = 128


def _sparsemax_block(x_ref, o_ref):
    zmax = jnp.max(x_ref[...], axis=-1, keepdims=True)
    lo = zmax - 1.0
    hi = zmax
    f_lo = jnp.sum(jnp.maximum(x_ref[...] - lo, 0.0), axis=-1,
                   keepdims=True) - 1.0
    t_p = lo - 1.0
    f_p = f_lo + 1.0

    def step(_, carry):
        lo, hi, f_lo, t_p, f_p = carry
        mid = 0.5 * (lo + hi)
        sec = lo + f_lo * (lo - t_p) / jnp.maximum(f_p - f_lo, 1e-30)
        t = jnp.where(sec < hi, jnp.maximum(sec, mid), mid)
        ft = jnp.sum(jnp.maximum(x_ref[...] - t, 0.0), axis=-1,
                     keepdims=True) - 1.0
        ge = ft >= 0.0
        return (
            jnp.where(ge, t, lo),
            jnp.where(ge, hi, t),
            jnp.where(ge, ft, f_lo),
            jnp.where(ge, lo, t_p),
            jnp.where(ge, f_lo, f_p),
        )

    lo, hi, f_lo, t_p, f_p = jax.lax.fori_loop(
        0, _SOLVE_ITERS, step, (lo, hi, f_lo, t_p, f_p))

    cnt = jnp.sum((x_ref[...] > lo).astype(jnp.float32), axis=-1,
                  keepdims=True)
    tau = lo + f_lo / jnp.maximum(cnt, 1.0)
    o_ref[...] = jnp.maximum(x_ref[...] - tau, 0.0)


@jax.jit
def kernel(input):
    n_rows, d = input.shape
    grid = (n_rows // _ROW_BLOCK,)
    return pl.pallas_call(
        _sparsemax_block,
        grid=grid,
        in_specs=[pl.BlockSpec((_ROW_BLOCK, d), lambda i: (i, 0))],
        out_specs=pl.BlockSpec((_ROW_BLOCK, d), lambda i: (i, 0)),
        out_shape=jax.ShapeDtypeStruct((n_rows, d), input.dtype),
        compiler_params=pltpu.CompilerParams(
            dimension_semantics=("parallel",),
        ),
    )(input)


# speedup vs baseline: 1.1665x; 1.1665x over previous
import jax
import jax.numpy as jnp
from jax.experimental import pallas as pl
from jax.experimental.pallas import tpu as pltpu

_SOLVE_ITERS = 7
_ROW_BLOCK = 64


def _sparsemax_block(x_ref, o_ref):
    zmax = jnp.max(x_ref[...], axis=-1, keepdims=True)
    lo = zmax - 1.0
    hi = zmax
    f_lo = jnp.sum(jnp.maximum(x_ref[...] - lo, 0.0), axis=-1,
                   keepdims=True) - 1.0
    t_p = lo - 1.0
    f_p = f_lo + 1.0

    def step(_, carry):
        lo, hi, f_lo, t_p, f_p = carry
        mid = 0.5 * (lo + hi)
        sec = lo + f_lo * (lo - t_p) / jnp.maximum(f_p - f_lo, 1e-30)
        t = jnp.where(sec < hi, jnp.maximum(sec, mid), mid)
        ft = jnp.sum(jnp.maximum(x_ref[...] - t, 0.0), axis=-1,
                     keepdims=True) - 1.0
        ge = ft >= 0.0
        return (
            jnp.where(ge, t, lo),
            jnp.where(ge, hi, t),
            jnp.where(ge, ft, f_lo),
            jnp.where(ge, lo, t_p),
            jnp.where(ge, f_lo, f_p),
        )

    lo, hi, f_lo, t_p, f_p = jax.lax.fori_loop(
        0, _SOLVE_ITERS, step, (lo, hi, f_lo, t_p, f_p))

    cnt = jnp.sum((x_ref[...] > lo).astype(jnp.float32), axis=-1,
                  keepdims=True)
    tau = lo + f_lo / jnp.maximum(cnt, 1.0)
    o_ref[...] = jnp.maximum(x_ref[...] - tau, 0.0)


@jax.jit
def kernel(input):
    n_rows, d = input.shape
    grid = (n_rows // _ROW_BLOCK,)
    return pl.pallas_call(
        _sparsemax_block,
        grid=grid,
        in_specs=[pl.BlockSpec((_ROW_BLOCK, d), lambda i: (i, 0))],
        out_specs=pl.BlockSpec((_ROW_BLOCK, d), lambda i: (i, 0)),
        out_shape=jax.ShapeDtypeStruct((n_rows, d), input.dtype),
        compiler_params=pltpu.CompilerParams(
            dimension_semantics=("parallel",),
        ),
    )(input)
